# Initial kernel scaffold; baseline (speedup 1.0000x reference)
import jax
import jax.numpy as jnp
from jax.experimental import pallas as pl


def kernel(x, table):
    raise NotImplementedError("write your pallas kernel here")


# speedup vs baseline: 4.0272x; 4.0272x over previous
import jax
import jax.numpy as jnp
from jax.experimental import pallas as pl
from jax.experimental.pallas import tpu as pltpu

_SEP = 102


def _body(x_ref, tab_ref, out_ref, t_ref):
    i = pl.program_id(0)

    @pl.when(i == 0)
    def _():
        r, s = x_ref.shape
        pos = (jax.lax.broadcasted_iota(jnp.int32, (r, s), 0) * s
               + jax.lax.broadcasted_iota(jnp.int32, (r, s), 1))
        sep = x_ref[...] == _SEP
        t_ref[0] = jnp.min(jnp.where(sep, pos, r * s))

    t = t_ref[0]
    blk, d = out_ref.shape
    row = i * blk + jax.lax.broadcasted_iota(jnp.int32, (blk, d), 0)
    t0 = jnp.broadcast_to(tab_ref[0:1, :], (blk, d))
    t1 = jnp.broadcast_to(tab_ref[1:2, :], (blk, d))
    out_ref[...] = jnp.where(row >= t, t1, t0)


def kernel(x, table):
    n = x.size
    d = table.shape[1]
    blk = 1024
    xr = x.reshape(32, n // 32)
    out = pl.pallas_call(
        _body,
        grid=(n // blk,),
        in_specs=[
            pl.BlockSpec(xr.shape, lambda i: (0, 0)),
            pl.BlockSpec(table.shape, lambda i: (0, 0)),
        ],
        out_specs=pl.BlockSpec((blk, d), lambda i: (i, 0)),
        out_shape=jax.ShapeDtypeStruct((n, d), table.dtype),
        scratch_shapes=[pltpu.SMEM((1,), jnp.int32)],
    )(xr, table)
    return out.reshape(x.shape + (d,))
